# baseline (speedup 1.0000x reference)
import functools

import jax
import jax.numpy as jnp
from jax import lax
from jax.experimental import pallas as pl
from jax.experimental.pallas import tpu as pltpu
from jax.experimental.pallas import tpu_sc as plsc

_N_CORES = 2
_N_SUBCORES = 16
_N_WORKERS = _N_CORES * _N_SUBCORES


def kernel(label, partition_matrix):
    (batch,) = label.shape
    n_cls, n_env = partition_matrix.shape
    b_per_w = batch // _N_WORKERS

    mesh = plsc.VectorSubcoreMesh(core_axis_name="c", subcore_axis_name="s")

    cp = pltpu.CompilerParams(
        needs_layout_passes=False, use_tc_tiling_on_sc=False
    )

    @functools.partial(
        pl.kernel,
        compiler_params=cp,
        out_type=jax.ShapeDtypeStruct((batch, n_env), jnp.float32),
        mesh=mesh,
        scratch_types=[
            pltpu.VMEM((b_per_w,), jnp.int32),
            pltpu.VMEM((b_per_w, n_env), jnp.float32),
            pltpu.SemaphoreType.DMA,
        ],
    )
    def _sc_kernel(label_hbm, table_hbm, out_hbm, idx_v, rows_v, sem):
        wid = lax.axis_index("s") * _N_CORES + lax.axis_index("c")
        base = wid * b_per_w

        pltpu.sync_copy(label_hbm.at[pl.ds(base, b_per_w)], idx_v)
        pltpu.async_copy(table_hbm.at[idx_v], rows_v, sem).wait()

        @pl.loop(0, b_per_w)
        def _(i):
            row = rows_v[i]
            m = jnp.max(row)
            e = jnp.exp(row - m)
            s = jnp.sum(e)
            rows_v[i] = e / s

        pltpu.sync_copy(rows_v, out_hbm.at[pl.ds(base, b_per_w)])

    return _sc_kernel(label, partition_matrix)


# speedup vs baseline: 1.0247x; 1.0247x over previous
import functools

import jax
import jax.numpy as jnp
from jax import lax
from jax.experimental import pallas as pl
from jax.experimental.pallas import tpu as pltpu
from jax.experimental.pallas import tpu_sc as plsc

_N_CORES = 2
_N_SUBCORES = 16
_N_WORKERS = _N_CORES * _N_SUBCORES
_LANES = 16


def kernel(label, partition_matrix):
    (batch,) = label.shape
    n_cls, n_env = partition_matrix.shape
    rows_per_group = 128 // n_env
    b_per_w = batch // _N_WORKERS
    chunks = b_per_w // _LANES

    table128 = partition_matrix.reshape(n_cls // rows_per_group, 128)

    mesh = plsc.VectorSubcoreMesh(core_axis_name="c", subcore_axis_name="s")

    cp = pltpu.CompilerParams(
        needs_layout_passes=False,
        use_tc_tiling_on_sc=True,
    )

    @functools.partial(
        pl.kernel,
        compiler_params=cp,
        out_type=jax.ShapeDtypeStruct((_N_WORKERS, b_per_w * n_env), jnp.float32),
        mesh=mesh,
        scratch_types=[
            pltpu.VMEM((b_per_w,), jnp.int32),
            pltpu.VMEM((b_per_w,), jnp.int32),
            pltpu.VMEM((b_per_w, 128), jnp.float32),
            pltpu.VMEM((b_per_w * n_env,), jnp.float32),
            pltpu.SemaphoreType.DMA,
        ],
    )
    def _sc_kernel(label_hbm, table_hbm, out_hbm, idx_v, grp_v, g_v, out_v, sem):
        wid = lax.axis_index("s") * _N_CORES + lax.axis_index("c")
        base = wid * b_per_w

        pltpu.sync_copy(label_hbm.at[pl.ds(base, b_per_w)], idx_v)

        @pl.loop(0, chunks)
        def _(c):
            sl = pl.ds(c * _LANES, _LANES)
            grp_v[sl] = jax.lax.shift_right_logical(idx_v[sl], rows_per_group.bit_length() - 1)

        pltpu.async_copy(table_hbm.at[grp_v], g_v, sem).wait()

        lane_iota = lax.iota(jnp.int32, _LANES)

        @pl.loop(0, chunks)
        def _(c):
            row0 = c * _LANES
            rows = lane_iota + row0
            lo = (idx_v[pl.ds(row0, _LANES)] & (rows_per_group - 1)) * n_env
            cols = [
                plsc.load_gather(g_v, [rows, lo + e]) for e in range(n_env)
            ]
            m = cols[0]
            for e in range(1, n_env):
                m = jnp.maximum(m, cols[e])
            exps = [jnp.exp(v - m) for v in cols]
            s = exps[0]
            for e in range(1, n_env):
                s = s + exps[e]
            inv = 1.0 / s
            obase = rows * n_env
            for e in range(n_env):
                plsc.store_scatter(out_v, [obase + e], exps[e] * inv)

        pltpu.sync_copy(out_v, out_hbm.at[wid])

    out = _sc_kernel(label, table128)
    return out.reshape(batch, n_env)


# speedup vs baseline: 6.2620x; 6.1110x over previous
import functools

import jax
import jax.numpy as jnp
from jax import lax
from jax.experimental import pallas as pl
from jax.experimental.pallas import tpu as pltpu
from jax.experimental.pallas import tpu_sc as plsc

_N_CORES = 2
_N_SUBCORES = 16
_N_WORKERS = _N_CORES * _N_SUBCORES
_LANES = 16
_TILE_W = 128


def kernel(label, partition_matrix):
    (batch,) = label.shape
    n_cls, n_env = partition_matrix.shape
    b_per_w = batch // _N_WORKERS
    n_chunks = b_per_w // _LANES

    table_t = jnp.swapaxes(partition_matrix, 0, 1)

    mesh = plsc.VectorSubcoreMesh(core_axis_name="c", subcore_axis_name="s")

    cp = pltpu.CompilerParams(
        needs_layout_passes=False,
        use_tc_tiling_on_sc=True,
    )

    @functools.partial(
        pl.kernel,
        compiler_params=cp,
        out_type=jax.ShapeDtypeStruct((n_env, batch), jnp.float32),
        mesh=mesh,
        scratch_types=[
            pltpu.VMEM((b_per_w,), jnp.int32),
            pltpu.VMEM((_LANES, n_env, _TILE_W), jnp.float32),
            pltpu.VMEM((_LANES, n_env, _TILE_W), jnp.float32),
            pltpu.VMEM((n_env, b_per_w), jnp.float32),
            pltpu.SemaphoreType.DMA,
            pltpu.SemaphoreType.DMA,
            pltpu.SemaphoreType.DMA,
        ],
    )
    def _sc_kernel(
        label_hbm, table_hbm, out_hbm,
        idx_v, buf_a, buf_b, res_v, sem_a, sem_b, sem_o,
    ):
        wid = lax.axis_index("s") * _N_CORES + lax.axis_index("c")
        base = wid * b_per_w

        pltpu.sync_copy(label_hbm.at[pl.ds(base, b_per_w)], idx_v)

        lane_iota = lax.iota(jnp.int32, _LANES)

        def fire(c, buf, sem):
            blks = (idx_v[pl.ds(c * _LANES, _LANES)] // _TILE_W) * _TILE_W
            for j in range(_LANES):
                blk = pl.multiple_of(blks[j], _TILE_W)
                pltpu.async_copy(
                    table_hbm.at[:, pl.ds(blk, _TILE_W)], buf.at[j], sem
                )

        def drain(buf, sem):
            @pl.loop(0, _LANES)
            def _(j):
                pltpu.make_async_copy(
                    table_hbm.at[:, pl.ds(0, _TILE_W)], buf.at[j], sem
                ).wait()

        def process(c, buf):
            lo = idx_v[pl.ds(c * _LANES, _LANES)] & (_TILE_W - 1)
            cols = [
                plsc.load_gather(
                    buf, [lane_iota, jnp.full((_LANES,), e, jnp.int32), lo]
                )
                for e in range(n_env)
            ]
            m = cols[0]
            for e in range(1, n_env):
                m = jnp.maximum(m, cols[e])
            exps = [jnp.exp(v - m) for v in cols]
            s = exps[0]
            for e in range(1, n_env):
                s = s + exps[e]
            inv = 1.0 / s
            sl = pl.ds(c * _LANES, _LANES)
            for e in range(n_env):
                res_v[e, sl] = exps[e] * inv

        fire(0, buf_a, sem_a)

        @pl.loop(0, n_chunks // 2)
        def _(t):
            c0 = t * 2
            fire(c0 + 1, buf_b, sem_b)
            drain(buf_a, sem_a)
            process(c0, buf_a)

            @pl.when(c0 + 2 < n_chunks)
            def _():
                fire(c0 + 2, buf_a, sem_a)

            drain(buf_b, sem_b)
            process(c0 + 1, buf_b)

        pltpu.async_copy(res_v, out_hbm.at[:, pl.ds(base, b_per_w)], sem_o).wait()

    out_t = _sc_kernel(label, table_t)
    return jnp.swapaxes(out_t, 0, 1)


# speedup vs baseline: 6.5698x; 1.0492x over previous
import functools

import jax
import jax.numpy as jnp
from jax import lax
from jax.experimental import pallas as pl
from jax.experimental.pallas import tpu as pltpu
from jax.experimental.pallas import tpu_sc as plsc

_N_CORES = 2
_N_SUBCORES = 16
_N_WORKERS = _N_CORES * _N_SUBCORES
_LANES = 16
_TILE_W = 128


def kernel(label, partition_matrix):
    (batch,) = label.shape
    n_cls, n_env = partition_matrix.shape
    b_per_w = batch // _N_WORKERS
    n_chunks = b_per_w // _LANES

    table_t = jnp.swapaxes(partition_matrix, 0, 1)

    mesh = plsc.VectorSubcoreMesh(core_axis_name="c", subcore_axis_name="s")

    cp = pltpu.CompilerParams(
        needs_layout_passes=False,
        use_tc_tiling_on_sc=True,
        disable_bounds_checks=True,
    )

    @functools.partial(
        pl.kernel,
        compiler_params=cp,
        out_type=jax.ShapeDtypeStruct((n_env, batch), jnp.float32),
        mesh=mesh,
        scratch_types=[
            pltpu.VMEM((b_per_w,), jnp.int32),
            pltpu.VMEM((_LANES, n_env, _TILE_W), jnp.float32),
            pltpu.VMEM((_LANES, n_env, _TILE_W), jnp.float32),
            pltpu.VMEM((_LANES, n_env, _TILE_W), jnp.float32),
            pltpu.VMEM((n_env, b_per_w), jnp.float32),
            pltpu.SemaphoreType.DMA,
            pltpu.SemaphoreType.DMA,
            pltpu.SemaphoreType.DMA,
            pltpu.SemaphoreType.DMA,
        ],
    )
    def _sc_kernel(
        label_hbm, table_hbm, out_hbm,
        idx_v, buf_a, buf_b, buf_c, res_v, sem_a, sem_b, sem_c, sem_o,
    ):
        wid = lax.axis_index("s") * _N_CORES + lax.axis_index("c")
        base = wid * b_per_w

        pltpu.sync_copy(label_hbm.at[pl.ds(base, b_per_w)], idx_v)

        lane_iota = lax.iota(jnp.int32, _LANES)

        def fire(c, buf, sem):
            blks = (idx_v[pl.ds(c * _LANES, _LANES)] // _TILE_W) * _TILE_W
            for j in range(_LANES):
                blk = pl.multiple_of(blks[j], _TILE_W)
                pltpu.async_copy(
                    table_hbm.at[:, pl.ds(blk, _TILE_W)], buf.at[j], sem
                )

        def drain(buf, sem):
            @pl.loop(0, _LANES)
            def _(j):
                pltpu.make_async_copy(
                    table_hbm.at[:, pl.ds(0, _TILE_W)], buf.at[j], sem
                ).wait()

        def process(c, buf):
            lo = idx_v[pl.ds(c * _LANES, _LANES)] & (_TILE_W - 1)
            cols = [
                plsc.load_gather(
                    buf, [lane_iota, jnp.full((_LANES,), e, jnp.int32), lo]
                )
                for e in range(n_env)
            ]
            m = cols[0]
            for e in range(1, n_env):
                m = jnp.maximum(m, cols[e])
            exps = [jnp.exp(v - m) for v in cols]
            s = exps[0]
            for e in range(1, n_env):
                s = s + exps[e]
            inv = 1.0 / s
            sl = pl.ds(c * _LANES, _LANES)
            for e in range(n_env):
                res_v[e, sl] = exps[e] * inv

        fire(0, buf_a, sem_a)
        fire(1, buf_b, sem_b)

        @pl.loop(0, n_chunks // 3)
        def _(t):
            c0 = t * 3
            fire(c0 + 2, buf_c, sem_c)
            drain(buf_a, sem_a)
            process(c0, buf_a)

            @pl.when(c0 + 3 < n_chunks)
            def _():
                fire(c0 + 3, buf_a, sem_a)

            drain(buf_b, sem_b)
            process(c0 + 1, buf_b)

            @pl.when(c0 + 4 < n_chunks)
            def _():
                fire(c0 + 4, buf_b, sem_b)

            drain(buf_c, sem_c)
            process(c0 + 2, buf_c)

        drain(buf_a, sem_a)
        process(n_chunks - 2, buf_a)
        drain(buf_b, sem_b)
        process(n_chunks - 1, buf_b)

        pltpu.async_copy(res_v, out_hbm.at[:, pl.ds(base, b_per_w)], sem_o).wait()

    out_t = _sc_kernel(label, table_t)
    return jnp.swapaxes(out_t, 0, 1)
